# baseline (speedup 1.0000x reference)
import jax
import jax.numpy as jnp
import numpy as np
from jax.experimental import pallas as pl
from jax.experimental.pallas import tpu as pltpu

ROWS_PER_FRAME = 543
N_FRAMES = 4096
MAX_LEN = 2048

_REF = [500, 501, 512, 513, 159, 386, 13]
_LIP = [61, 185, 40, 39, 37, 0, 267, 269, 270, 409, 291, 146, 91, 181, 84,
        17, 314, 405, 321, 375, 78, 191, 80, 81, 82, 13, 312, 311, 310, 415,
        95, 88, 178, 87, 14, 317, 402, 318, 324, 308]
_LLIP = _LIP[10::-1] + _LIP[19:10:-1] + _LIP[29:19:-1] + _LIP[39:29:-1]
_LHAND = list(range(468, 489))
_RHAND = list(range(522, 543))

_SEL_R = _LIP + _RHAND
_SEL_L = _LLIP + _LHAND

_RB = 8
_STAT_ROWBLKS = sorted({lm // _RB for lm in _REF + _LHAND + _RHAND})
_NRB = len(_STAT_ROWBLKS)

_SEL_ROWBLKS = sorted({lm // _RB for lm in set(_SEL_R) | set(_SEL_L)})
_NSB = len(_SEL_ROWBLKS)


def _build_sel():
    S0 = np.zeros((128, ROWS_PER_FRAME), np.float32)
    S1 = np.zeros((128, ROWS_PER_FRAME), np.float32)
    Sm = np.zeros((128, ROWS_PER_FRAME), np.float32)
    for j, lm in enumerate(_SEL_R):
        S0[j, lm] = 1.0
        S1[j, lm] = 1.0
        Sm[j, lm] = 1.0
    for j, lm in enumerate(_SEL_L):
        S0[64 + j, lm] = -1.0
        S1[64 + j, lm] = 1.0
        Sm[64 + j, lm] = 1.0
    pad = ((ROWS_PER_FRAME + _RB - 1) // _RB) * _RB - ROWS_PER_FRAME
    S0, S1, Sm = (np.pad(S, ((0, 0), (0, pad))) for S in (S0, S1, Sm))
    def pack(S):
        blocks = np.stack(
            [S[:, b * _RB:(b + 1) * _RB] for b in _SEL_ROWBLKS])
        return jnp.asarray(blocks)

    return pack(S0), pack(S1), pack(Sm)


def _build_stat_masks():
    m = np.zeros((_NRB, 3, _RB), np.float32)
    for i, blk in enumerate(_STAT_ROWBLKS):
        for r in range(_RB):
            lm = blk * _RB + r
            if lm in _REF:
                m[i, 0, r] = 1.0
            if lm in _LHAND:
                m[i, 1, r] = 1.0
            if lm in _RHAND:
                m[i, 2, r] = 1.0
    return jnp.asarray(m)


def _rowblk(rb):
    v = 55 + rb + jnp.where(rb >= 8, 1, 0)
    v = jnp.where(rb == 0, 1, v)
    v = jnp.where(rb == 1, 19, v)
    v = jnp.where(rb == 2, 48, v)
    return v


def _stats_body(x_ref, mask_ref, out_ref, cnt_ref):
    fb = pl.program_id(0)
    rb = pl.program_id(1)
    x = x_ref[...]
    isn = jnp.isnan(x)
    nanany = (isn[0] | isn[1] | isn[2]).astype(jnp.float32)
    x0 = jnp.where(isn, 0.0, x)

    refw = mask_ref[0, 0, :][:, None]
    lw = mask_ref[0, 1, :][:, None]
    rw = mask_ref[0, 2, :][:, None]

    w = refw * (1.0 - nanany)
    cnt = jnp.sum(w)
    sums = []
    sumsq = []
    for c in range(3):
        v = x0[c] * w
        sums.append(jnp.sum(v))
        sumsq.append(jnp.sum(v * x0[c]))

    lane = jax.lax.broadcasted_iota(jnp.int32, (1, 128), 1)
    part = jnp.zeros((1, 128), jnp.float32)
    part = jnp.where(lane == 0, cnt, part)
    for c in range(3):
        part = jnp.where(lane == 1 + c, sums[c], part)
        part = jnp.where(lane == 4 + c, sumsq[c], part)

    @pl.when((fb == 0) & (rb == 0))
    def _():
        out_ref[...] = jnp.zeros_like(out_ref)

    out_ref[...] += part

    lpart = jnp.sum(lw * nanany, axis=0, keepdims=True)
    rpart = jnp.sum(rw * nanany, axis=0, keepdims=True)

    @pl.when(rb == 0)
    def _():
        cnt_ref[...] = jnp.zeros_like(cnt_ref)

    cnt_ref[0:1, :] += lpart
    cnt_ref[1:2, :] += rpart


def _main_body(blk_ref, part_ref, hand_ref, x_ref, s0_ref, s1_ref, sm_ref,
               e_ref, out_ref, acc0, acc1, accm):
    rb = pl.program_id(1)

    x = x_ref[...]
    isn = jnp.isnan(x)
    nanany = (isn[0] | isn[1] | isn[2]).astype(jnp.float32)
    x0 = jnp.where(isn, 0.0, x)

    va = jnp.dot(s0_ref[0], x0[0], preferred_element_type=jnp.float32)
    vb = jnp.dot(s1_ref[0], x0[1], preferred_element_type=jnp.float32)
    mk = jnp.dot(sm_ref[0], nanany, preferred_element_type=jnp.float32)

    @pl.when(rb == 0)
    def _():
        acc0[...] = jnp.zeros_like(acc0)
        acc1[...] = jnp.zeros_like(acc1)
        accm[...] = jnp.zeros_like(accm)

    acc0[...] += va
    acc1[...] += vb
    accm[...] += mk

    @pl.when(rb == _NSB - 1)
    def _():
        _main_tail(part_ref, hand_ref, e_ref, out_ref, acc0, acc1, accm)


def _main_tail(part_ref, hand_ref, e_ref, out_ref, acc0, acc1, accm):
    p = part_ref[...]
    cnt = p[0, 0]
    m0 = p[0, 1] / cnt
    m1 = p[0, 2] / cnt
    m2 = p[0, 3] / cnt
    v0 = p[0, 4] / cnt - m0 * m0
    v1 = p[0, 5] / cnt - m1 * m1
    v2 = p[0, 6] / cnt - m2 * m2
    inv_s = 3.0 / (jnp.sqrt(v0) + jnp.sqrt(v1) + jnp.sqrt(v2))
    hn = hand_ref[...]
    lcnt = jnp.sum((hn[0:1, :] == 0.0).astype(jnp.float32))
    rcnt = jnp.sum((hn[1:2, :] == 0.0).astype(jnp.float32))
    lhanded = lcnt > rcnt

    e = e_ref[...]
    va = jnp.dot(acc0[...], e, preferred_element_type=jnp.float32)
    vb = jnp.dot(acc1[...], e, preferred_element_type=jnp.float32)
    mk = jnp.dot(accm[...], e, preferred_element_type=jnp.float32)

    val0 = jnp.where(lhanded, va[64:128, :], va[0:64, :])
    val1 = jnp.where(lhanded, vb[64:128, :], vb[0:64, :])
    bad = jnp.where(lhanded, mk[64:128, :], mk[0:64, :]) > 0.5
    sgn0 = jnp.where(lhanded, -1.0, 1.0)

    r0 = (val0 - sgn0 * m0) * inv_s
    r1 = (val1 - m1) * inv_s
    r0 = jnp.where(bad, 0.0, r0)
    r1 = jnp.where(bad, 0.0, r1)
    out_ref[0, :, :] = r0
    out_ref[1, :, :] = r1


@jax.jit
def kernel(frames):
    S0, S1, Sm = _build_sel()
    masks = _build_stat_masks()
    xT = jnp.transpose(frames, (2, 1, 0))

    FA = 1024
    partials, handcnt = pl.pallas_call(
        _stats_body,
        grid=(N_FRAMES // FA, _NRB),
        in_specs=[
            pl.BlockSpec((3, _RB, FA), lambda fb, rb: (0, _rowblk(rb), fb)),
            pl.BlockSpec((1, 3, _RB), lambda fb, rb: (rb, 0, 0)),
        ],
        out_specs=[
            pl.BlockSpec((1, 128), lambda fb, rb: (0, 0)),
            pl.BlockSpec((2, FA), lambda fb, rb: (0, fb)),
        ],
        out_shape=[
            jax.ShapeDtypeStruct((1, 128), jnp.float32),
            jax.ShapeDtypeStruct((2, N_FRAMES), jnp.float32),
        ],
    )(xT, masks)

    FB = 512
    E = np.zeros((FB, FB // 2), np.float32)
    E[np.arange(0, FB, 2), np.arange(FB // 2)] = 1.0
    E = jnp.asarray(E)

    sel_blks = jnp.asarray(_SEL_ROWBLKS, jnp.int32)
    full = pl.pallas_call(
        _main_body,
        grid_spec=pltpu.PrefetchScalarGridSpec(
            num_scalar_prefetch=1,
            grid=(N_FRAMES // FB, _NSB),
            in_specs=[
                pl.BlockSpec((1, 128), lambda fb, rb, blks: (0, 0)),
                pl.BlockSpec((2, N_FRAMES), lambda fb, rb, blks: (0, 0)),
                pl.BlockSpec((3, _RB, FB),
                             lambda fb, rb, blks: (0, blks[rb], fb)),
                pl.BlockSpec((1, 128, _RB), lambda fb, rb, blks: (rb, 0, 0)),
                pl.BlockSpec((1, 128, _RB), lambda fb, rb, blks: (rb, 0, 0)),
                pl.BlockSpec((1, 128, _RB), lambda fb, rb, blks: (rb, 0, 0)),
                pl.BlockSpec((FB, FB // 2), lambda fb, rb, blks: (0, 0)),
            ],
            out_specs=pl.BlockSpec((2, 64, FB // 2),
                                   lambda fb, rb, blks: (0, 0, fb)),
            scratch_shapes=[
                pltpu.VMEM((128, FB), jnp.float32),
                pltpu.VMEM((128, FB), jnp.float32),
                pltpu.VMEM((128, FB), jnp.float32),
            ],
        ),
        out_shape=jax.ShapeDtypeStruct((2, 64, MAX_LEN), jnp.float32),
    )(sel_blks, partials, handcnt, xT, S0, S1, Sm, E)

    res = full[:, :61, :]
    return jnp.transpose(res, (2, 1, 0))


# speedup vs baseline: 4.2906x; 4.2906x over previous
import jax
import jax.numpy as jnp
import numpy as np
from jax.experimental import pallas as pl
from jax.experimental.pallas import tpu as pltpu

ROWS_PER_FRAME = 543
N_FRAMES = 4096
MAX_LEN = 2048

_REF = [500, 501, 512, 513, 159, 386, 13]
_LIP = [61, 185, 40, 39, 37, 0, 267, 269, 270, 409, 291, 146, 91, 181, 84,
        17, 314, 405, 321, 375, 78, 191, 80, 81, 82, 13, 312, 311, 310, 415,
        95, 88, 178, 87, 14, 317, 402, 318, 324, 308]
_LLIP = _LIP[10::-1] + _LIP[19:10:-1] + _LIP[29:19:-1] + _LIP[39:29:-1]
_LHAND = list(range(468, 489))
_RHAND = list(range(522, 543))

_SEL_R = _LIP + _RHAND
_SEL_L = _LLIP + _LHAND

_RB = 8
_STAT_ROWBLKS = sorted({lm // _RB for lm in _REF + _LHAND + _RHAND})
_NRB = len(_STAT_ROWBLKS)

_SEL_ROWBLKS = sorted({lm // _RB for lm in set(_SEL_R) | set(_SEL_L)})
_NSB = len(_SEL_ROWBLKS)


def _build_sel():
    S0 = np.zeros((128, ROWS_PER_FRAME), np.float32)
    S1 = np.zeros((128, ROWS_PER_FRAME), np.float32)
    Sm = np.zeros((128, ROWS_PER_FRAME), np.float32)
    for j, lm in enumerate(_SEL_R):
        S0[j, lm] = 1.0
        S1[j, lm] = 1.0
        Sm[j, lm] = 1.0
    for j, lm in enumerate(_SEL_L):
        S0[64 + j, lm] = -1.0
        S1[64 + j, lm] = 1.0
        Sm[64 + j, lm] = 1.0
    pad = ((ROWS_PER_FRAME + _RB - 1) // _RB) * _RB - ROWS_PER_FRAME
    S0, S1, Sm = (np.pad(S, ((0, 0), (0, pad))) for S in (S0, S1, Sm))
    def pack(S):
        return jnp.asarray(np.concatenate(
            [S[:, b * _RB:(b + 1) * _RB] for b in _SEL_ROWBLKS], axis=1))

    return pack(S0), pack(S1), pack(Sm)


def _build_stat_masks():
    m = np.zeros((_NRB, 3, _RB), np.float32)
    for i, blk in enumerate(_STAT_ROWBLKS):
        for r in range(_RB):
            lm = blk * _RB + r
            if lm in _REF:
                m[i, 0, r] = 1.0
            if lm in _LHAND:
                m[i, 1, r] = 1.0
            if lm in _RHAND:
                m[i, 2, r] = 1.0
    return jnp.asarray(m)


def _rowblk(rb):
    v = 55 + rb + jnp.where(rb >= 8, 1, 0)
    v = jnp.where(rb == 0, 1, v)
    v = jnp.where(rb == 1, 19, v)
    v = jnp.where(rb == 2, 48, v)
    return v


def _stats_body(x_ref, mask_ref, out_ref, cnt_ref):
    fb = pl.program_id(0)
    rb = pl.program_id(1)
    x = x_ref[...]
    isn = jnp.isnan(x)
    nanany = (isn[0] | isn[1] | isn[2]).astype(jnp.float32)
    x0 = jnp.where(isn, 0.0, x)

    refw = mask_ref[0, 0, :][:, None]
    lw = mask_ref[0, 1, :][:, None]
    rw = mask_ref[0, 2, :][:, None]

    w = refw * (1.0 - nanany)
    cnt = jnp.sum(w)
    sums = []
    sumsq = []
    for c in range(3):
        v = x0[c] * w
        sums.append(jnp.sum(v))
        sumsq.append(jnp.sum(v * x0[c]))

    lane = jax.lax.broadcasted_iota(jnp.int32, (1, 128), 1)
    part = jnp.zeros((1, 128), jnp.float32)
    part = jnp.where(lane == 0, cnt, part)
    for c in range(3):
        part = jnp.where(lane == 1 + c, sums[c], part)
        part = jnp.where(lane == 4 + c, sumsq[c], part)

    @pl.when((fb == 0) & (rb == 0))
    def _():
        out_ref[...] = jnp.zeros_like(out_ref)

    out_ref[...] += part

    lpart = jnp.sum(lw * nanany, axis=0, keepdims=True)
    rpart = jnp.sum(rw * nanany, axis=0, keepdims=True)

    @pl.when(rb == 0)
    def _():
        cnt_ref[...] = jnp.zeros_like(cnt_ref)

    cnt_ref[0:1, :] += lpart
    cnt_ref[1:2, :] += rpart


def _main_body(part_ref, hand_ref, *refs):
    xrefs = refs[:_NSB]
    s0_ref, s1_ref, sm_ref, e_ref, out_ref = refs[_NSB:]

    x = jnp.concatenate([r[...] for r in xrefs], axis=1)
    isn = jnp.isnan(x)
    nanany = (isn[0] | isn[1] | isn[2]).astype(jnp.float32)
    x0 = jnp.where(isn, 0.0, x)

    acc0 = jnp.dot(s0_ref[...], x0[0], preferred_element_type=jnp.float32)
    acc1 = jnp.dot(s1_ref[...], x0[1], preferred_element_type=jnp.float32)
    accm = jnp.dot(sm_ref[...], nanany, preferred_element_type=jnp.float32)

    p = part_ref[...]
    cnt = p[0, 0]
    m0 = p[0, 1] / cnt
    m1 = p[0, 2] / cnt
    m2 = p[0, 3] / cnt
    v0 = p[0, 4] / cnt - m0 * m0
    v1 = p[0, 5] / cnt - m1 * m1
    v2 = p[0, 6] / cnt - m2 * m2
    inv_s = 3.0 / (jnp.sqrt(v0) + jnp.sqrt(v1) + jnp.sqrt(v2))
    hn = hand_ref[...]
    lcnt = jnp.sum((hn[0:1, :] == 0.0).astype(jnp.float32))
    rcnt = jnp.sum((hn[1:2, :] == 0.0).astype(jnp.float32))
    lhanded = lcnt > rcnt

    e = e_ref[...]
    va = jnp.dot(acc0, e, preferred_element_type=jnp.float32)
    vb = jnp.dot(acc1, e, preferred_element_type=jnp.float32)
    mk = jnp.dot(accm, e, preferred_element_type=jnp.float32)

    val0 = jnp.where(lhanded, va[64:128, :], va[0:64, :])
    val1 = jnp.where(lhanded, vb[64:128, :], vb[0:64, :])
    bad = jnp.where(lhanded, mk[64:128, :], mk[0:64, :]) > 0.5
    sgn0 = jnp.where(lhanded, -1.0, 1.0)

    r0 = (val0 - sgn0 * m0) * inv_s
    r1 = (val1 - m1) * inv_s
    r0 = jnp.where(bad, 0.0, r0)
    r1 = jnp.where(bad, 0.0, r1)
    out_ref[0, :, :] = r0
    out_ref[1, :, :] = r1


@jax.jit
def kernel(frames):
    S0, S1, Sm = _build_sel()
    masks = _build_stat_masks()
    xT = jnp.transpose(frames, (2, 1, 0))

    FA = 1024
    partials, handcnt = pl.pallas_call(
        _stats_body,
        grid=(N_FRAMES // FA, _NRB),
        in_specs=[
            pl.BlockSpec((3, _RB, FA), lambda fb, rb: (0, _rowblk(rb), fb)),
            pl.BlockSpec((1, 3, _RB), lambda fb, rb: (rb, 0, 0)),
        ],
        out_specs=[
            pl.BlockSpec((1, 128), lambda fb, rb: (0, 0)),
            pl.BlockSpec((2, FA), lambda fb, rb: (0, fb)),
        ],
        out_shape=[
            jax.ShapeDtypeStruct((1, 128), jnp.float32),
            jax.ShapeDtypeStruct((2, N_FRAMES), jnp.float32),
        ],
    )(xT, masks)

    FB = 512
    E = np.zeros((FB, FB // 2), np.float32)
    E[np.arange(0, FB, 2), np.arange(FB // 2)] = 1.0
    E = jnp.asarray(E)

    x_specs = [
        pl.BlockSpec((3, _RB, FB), lambda j, b=b: (0, b, j))
        for b in _SEL_ROWBLKS
    ]
    full = pl.pallas_call(
        _main_body,
        grid=(N_FRAMES // FB,),
        in_specs=[
            pl.BlockSpec((1, 128), lambda j: (0, 0)),
            pl.BlockSpec((2, N_FRAMES), lambda j: (0, 0)),
            *x_specs,
            pl.BlockSpec((128, _NSB * _RB), lambda j: (0, 0)),
            pl.BlockSpec((128, _NSB * _RB), lambda j: (0, 0)),
            pl.BlockSpec((128, _NSB * _RB), lambda j: (0, 0)),
            pl.BlockSpec((FB, FB // 2), lambda j: (0, 0)),
        ],
        out_specs=pl.BlockSpec((2, 64, FB // 2), lambda j: (0, 0, j)),
        out_shape=jax.ShapeDtypeStruct((2, 64, MAX_LEN), jnp.float32),
    )(partials, handcnt, *([xT] * _NSB), S0, S1, Sm, E)

    res = full[:, :61, :]
    return jnp.transpose(res, (2, 1, 0))


# speedup vs baseline: 10.2431x; 2.3874x over previous
import jax
import jax.numpy as jnp
import numpy as np
from jax.experimental import pallas as pl
from jax.experimental.pallas import tpu as pltpu

ROWS_PER_FRAME = 543
N_FRAMES = 4096
MAX_LEN = 2048

_REF = [500, 501, 512, 513, 159, 386, 13]
_LIP = [61, 185, 40, 39, 37, 0, 267, 269, 270, 409, 291, 146, 91, 181, 84,
        17, 314, 405, 321, 375, 78, 191, 80, 81, 82, 13, 312, 311, 310, 415,
        95, 88, 178, 87, 14, 317, 402, 318, 324, 308]
_LLIP = _LIP[10::-1] + _LIP[19:10:-1] + _LIP[29:19:-1] + _LIP[39:29:-1]
_LHAND = list(range(468, 489))
_RHAND = list(range(522, 543))

_SEL_R = _LIP + _RHAND
_SEL_L = _LLIP + _LHAND

_RB = 8
_STAT_ROWBLKS = sorted({lm // _RB for lm in _REF + _LHAND + _RHAND})
_NRB = len(_STAT_ROWBLKS)

_SEL_ROWBLKS = sorted({lm // _RB for lm in set(_SEL_R) | set(_SEL_L)})
_NSB = len(_SEL_ROWBLKS)


def _build_sel():
    S0 = np.zeros((128, ROWS_PER_FRAME), np.float32)
    S1 = np.zeros((128, ROWS_PER_FRAME), np.float32)
    Sm = np.zeros((128, ROWS_PER_FRAME), np.float32)
    for j, lm in enumerate(_SEL_R):
        S0[j, lm] = 1.0
        S1[j, lm] = 1.0
        Sm[j, lm] = 1.0
    for j, lm in enumerate(_SEL_L):
        S0[64 + j, lm] = -1.0
        S1[64 + j, lm] = 1.0
        Sm[64 + j, lm] = 1.0
    pad = ((ROWS_PER_FRAME + _RB - 1) // _RB) * _RB - ROWS_PER_FRAME
    S0, S1, Sm = (np.pad(S, ((0, 0), (0, pad))) for S in (S0, S1, Sm))
    def pack(S):
        return jnp.asarray(np.concatenate(
            [S[:, b * _RB:(b + 1) * _RB] for b in _SEL_ROWBLKS], axis=1))

    return pack(S0), pack(S1), pack(Sm)


def _build_stat_masks():
    n = _NRB * _RB
    m = np.zeros((3, n, 1), np.float32)
    for i, blk in enumerate(_STAT_ROWBLKS):
        for r in range(_RB):
            lm = blk * _RB + r
            row = i * _RB + r
            if lm in _REF:
                m[0, row, 0] = 1.0
            if lm in _LHAND:
                m[1, row, 0] = 1.0
            if lm in _RHAND:
                m[2, row, 0] = 1.0
    return m


def _stats_body(mask_ref, *refs):
    fb = pl.program_id(0)
    xrefs = refs[:_NRB]
    out_ref, cnt_ref = refs[_NRB:]

    x = jnp.concatenate([r[...] for r in xrefs], axis=1)
    isn = jnp.isnan(x)
    nanany = (isn[0] | isn[1] | isn[2]).astype(jnp.float32)
    x0 = jnp.where(isn, 0.0, x)

    refw = mask_ref[0]
    lw = mask_ref[1]
    rw = mask_ref[2]

    w = refw * (1.0 - nanany)
    cnt = jnp.sum(w)
    sums = []
    sumsq = []
    for c in range(3):
        v = x0[c] * w
        sums.append(jnp.sum(v))
        sumsq.append(jnp.sum(v * x0[c]))

    lane = jax.lax.broadcasted_iota(jnp.int32, (1, 128), 1)
    part = jnp.zeros((1, 128), jnp.float32)
    part = jnp.where(lane == 0, cnt, part)
    for c in range(3):
        part = jnp.where(lane == 1 + c, sums[c], part)
        part = jnp.where(lane == 4 + c, sumsq[c], part)

    @pl.when(fb == 0)
    def _():
        out_ref[...] = jnp.zeros_like(out_ref)

    out_ref[...] += part

    lpart = jnp.sum(lw * nanany, axis=0, keepdims=True)
    rpart = jnp.sum(rw * nanany, axis=0, keepdims=True)
    cnt_ref[...] = jnp.concatenate([lpart, rpart], axis=0)


def _main_body(part_ref, hand_ref, *refs):
    xrefs = refs[:_NSB]
    s0_ref, s1_ref, sm_ref, e_ref, out_ref = refs[_NSB:]

    x = jnp.concatenate([r[...] for r in xrefs], axis=1)
    isn = jnp.isnan(x)
    nanany = (isn[0] | isn[1] | isn[2]).astype(jnp.float32)
    x0 = jnp.where(isn, 0.0, x)

    acc0 = jnp.dot(s0_ref[...], x0[0], preferred_element_type=jnp.float32)
    acc1 = jnp.dot(s1_ref[...], x0[1], preferred_element_type=jnp.float32)
    accm = jnp.dot(sm_ref[...], nanany, preferred_element_type=jnp.float32)

    p = part_ref[...]
    cnt = p[0, 0]
    m0 = p[0, 1] / cnt
    m1 = p[0, 2] / cnt
    m2 = p[0, 3] / cnt
    v0 = p[0, 4] / cnt - m0 * m0
    v1 = p[0, 5] / cnt - m1 * m1
    v2 = p[0, 6] / cnt - m2 * m2
    inv_s = 3.0 / (jnp.sqrt(v0) + jnp.sqrt(v1) + jnp.sqrt(v2))
    hn = hand_ref[...]
    lcnt = jnp.sum((hn[0:1, :] == 0.0).astype(jnp.float32))
    rcnt = jnp.sum((hn[1:2, :] == 0.0).astype(jnp.float32))
    lhanded = lcnt > rcnt

    e = e_ref[...]
    va = jnp.dot(acc0, e, preferred_element_type=jnp.float32)
    vb = jnp.dot(acc1, e, preferred_element_type=jnp.float32)
    mk = jnp.dot(accm, e, preferred_element_type=jnp.float32)

    val0 = jnp.where(lhanded, va[64:128, :], va[0:64, :])
    val1 = jnp.where(lhanded, vb[64:128, :], vb[0:64, :])
    bad = jnp.where(lhanded, mk[64:128, :], mk[0:64, :]) > 0.5
    sgn0 = jnp.where(lhanded, -1.0, 1.0)

    r0 = (val0 - sgn0 * m0) * inv_s
    r1 = (val1 - m1) * inv_s
    r0 = jnp.where(bad, 0.0, r0)
    r1 = jnp.where(bad, 0.0, r1)
    out_ref[0, :, :] = r0
    out_ref[1, :, :] = r1


@jax.jit
def kernel(frames):
    S0, S1, Sm = _build_sel()
    xT = jnp.transpose(frames, (2, 1, 0))

    FA = 1024
    a_specs = [
        pl.BlockSpec((3, _RB, FA), lambda fb, b=b: (0, b, fb))
        for b in _STAT_ROWBLKS
    ]
    partials, handcnt = pl.pallas_call(
        _stats_body,
        grid=(N_FRAMES // FA,),
        in_specs=[
            pl.BlockSpec((3, _NRB * _RB, 1), lambda fb: (0, 0, 0)),
            *a_specs,
        ],
        out_specs=[
            pl.BlockSpec((1, 128), lambda fb: (0, 0)),
            pl.BlockSpec((2, FA), lambda fb: (0, fb)),
        ],
        out_shape=[
            jax.ShapeDtypeStruct((1, 128), jnp.float32),
            jax.ShapeDtypeStruct((2, N_FRAMES), jnp.float32),
        ],
    )(jnp.asarray(_build_stat_masks()), *([xT] * _NRB))

    FB = 1024
    E = np.zeros((FB, FB // 2), np.float32)
    E[np.arange(0, FB, 2), np.arange(FB // 2)] = 1.0
    E = jnp.asarray(E)

    x_specs = [
        pl.BlockSpec((3, _RB, FB), lambda j, b=b: (0, b, j))
        for b in _SEL_ROWBLKS
    ]
    full = pl.pallas_call(
        _main_body,
        grid=(N_FRAMES // FB,),
        in_specs=[
            pl.BlockSpec((1, 128), lambda j: (0, 0)),
            pl.BlockSpec((2, N_FRAMES), lambda j: (0, 0)),
            *x_specs,
            pl.BlockSpec((128, _NSB * _RB), lambda j: (0, 0)),
            pl.BlockSpec((128, _NSB * _RB), lambda j: (0, 0)),
            pl.BlockSpec((128, _NSB * _RB), lambda j: (0, 0)),
            pl.BlockSpec((FB, FB // 2), lambda j: (0, 0)),
        ],
        out_specs=pl.BlockSpec((2, 64, FB // 2), lambda j: (0, 0, j)),
        out_shape=jax.ShapeDtypeStruct((2, 64, MAX_LEN), jnp.float32),
    )(partials, handcnt, *([xT] * _NSB), S0, S1, Sm, E)

    res = full[:, :61, :]
    return jnp.transpose(res, (2, 1, 0))


# speedup vs baseline: 10.3137x; 1.0069x over previous
import jax
import jax.numpy as jnp
import numpy as np
from jax.experimental import pallas as pl
from jax.experimental.pallas import tpu as pltpu

ROWS_PER_FRAME = 543
N_FRAMES = 4096
MAX_LEN = 2048

_REF = [500, 501, 512, 513, 159, 386, 13]
_LIP = [61, 185, 40, 39, 37, 0, 267, 269, 270, 409, 291, 146, 91, 181, 84,
        17, 314, 405, 321, 375, 78, 191, 80, 81, 82, 13, 312, 311, 310, 415,
        95, 88, 178, 87, 14, 317, 402, 318, 324, 308]
_LLIP = _LIP[10::-1] + _LIP[19:10:-1] + _LIP[29:19:-1] + _LIP[39:29:-1]
_LHAND = list(range(468, 489))
_RHAND = list(range(522, 543))

_SEL_R = _LIP + _RHAND
_SEL_L = _LLIP + _LHAND

_RB = 8
_STAT_ROWBLKS = sorted({lm // _RB for lm in _REF + _LHAND + _RHAND})
_NRB = len(_STAT_ROWBLKS)

_SEL_ROWBLKS = sorted({lm // _RB for lm in set(_SEL_R) | set(_SEL_L)})
_NSB = len(_SEL_ROWBLKS)


def _build_sel():
    S0 = np.zeros((128, ROWS_PER_FRAME), np.float32)
    S1 = np.zeros((128, ROWS_PER_FRAME), np.float32)
    Sm = np.zeros((128, ROWS_PER_FRAME), np.float32)
    for j, lm in enumerate(_SEL_R):
        S0[j, lm] = 1.0
        S1[j, lm] = 1.0
        Sm[j, lm] = 1.0
    for j, lm in enumerate(_SEL_L):
        S0[64 + j, lm] = -1.0
        S1[64 + j, lm] = 1.0
        Sm[64 + j, lm] = 1.0
    pad = ((ROWS_PER_FRAME + _RB - 1) // _RB) * _RB - ROWS_PER_FRAME
    S0, S1, Sm = (np.pad(S, ((0, 0), (0, pad))) for S in (S0, S1, Sm))
    def pack(S):
        return jnp.asarray(np.concatenate(
            [S[:, b * _RB:(b + 1) * _RB] for b in _SEL_ROWBLKS], axis=1))

    return pack(S0), pack(S1), pack(Sm)


def _build_stat_masks():
    n = _NRB * _RB
    m = np.zeros((3, n, 1), np.float32)
    for i, blk in enumerate(_STAT_ROWBLKS):
        for r in range(_RB):
            lm = blk * _RB + r
            row = i * _RB + r
            if lm in _REF:
                m[0, row, 0] = 1.0
            if lm in _LHAND:
                m[1, row, 0] = 1.0
            if lm in _RHAND:
                m[2, row, 0] = 1.0
    return m


def _stats_body(mask_ref, *refs):
    fb = pl.program_id(0)
    xrefs = refs[:_NRB]
    out_ref, cnt_ref = refs[_NRB:]

    x = jnp.concatenate([r[...] for r in xrefs], axis=1)
    isn = jnp.isnan(x)
    nanany = (isn[0] | isn[1] | isn[2]).astype(jnp.float32)
    x0 = jnp.where(isn, 0.0, x)

    refw = mask_ref[0]
    lw = mask_ref[1]
    rw = mask_ref[2]

    w = refw * (1.0 - nanany)
    cnt = jnp.sum(w)
    sums = []
    sumsq = []
    for c in range(3):
        v = x0[c] * w
        sums.append(jnp.sum(v))
        sumsq.append(jnp.sum(v * x0[c]))

    lane = jax.lax.broadcasted_iota(jnp.int32, (1, 128), 1)
    part = jnp.zeros((1, 128), jnp.float32)
    part = jnp.where(lane == 0, cnt, part)
    for c in range(3):
        part = jnp.where(lane == 1 + c, sums[c], part)
        part = jnp.where(lane == 4 + c, sumsq[c], part)

    @pl.when(fb == 0)
    def _():
        out_ref[...] = jnp.zeros_like(out_ref)

    out_ref[...] += part

    lpart = jnp.sum(lw * nanany, axis=0, keepdims=True)
    rpart = jnp.sum(rw * nanany, axis=0, keepdims=True)
    cnt_ref[...] = jnp.concatenate([lpart, rpart], axis=0)


def _main_body(part_ref, hand_ref, *refs):
    xrefs = refs[:_NSB]
    s0_ref, s1_ref, sm_ref, e_ref, out_ref = refs[_NSB:]

    x = jnp.concatenate([r[...] for r in xrefs], axis=1)
    isn = jnp.isnan(x)
    nanany = (isn[0] | isn[1] | isn[2]).astype(jnp.float32)
    x0 = jnp.where(isn, 0.0, x)

    acc0 = jnp.dot(s0_ref[...], x0[0], preferred_element_type=jnp.float32)
    acc1 = jnp.dot(s1_ref[...], x0[1], preferred_element_type=jnp.float32)
    accm = jnp.dot(sm_ref[...], nanany, preferred_element_type=jnp.float32)

    p = part_ref[...]
    cnt = p[0, 0]
    m0 = p[0, 1] / cnt
    m1 = p[0, 2] / cnt
    m2 = p[0, 3] / cnt
    v0 = p[0, 4] / cnt - m0 * m0
    v1 = p[0, 5] / cnt - m1 * m1
    v2 = p[0, 6] / cnt - m2 * m2
    inv_s = 3.0 / (jnp.sqrt(v0) + jnp.sqrt(v1) + jnp.sqrt(v2))
    hn = hand_ref[...]
    lcnt = jnp.sum((hn[0:1, :] == 0.0).astype(jnp.float32))
    rcnt = jnp.sum((hn[1:2, :] == 0.0).astype(jnp.float32))
    lhanded = lcnt > rcnt

    e = e_ref[...]
    va = jnp.dot(acc0, e, preferred_element_type=jnp.float32)
    vb = jnp.dot(acc1, e, preferred_element_type=jnp.float32)
    mk = jnp.dot(accm, e, preferred_element_type=jnp.float32)

    val0 = jnp.where(lhanded, va[64:128, :], va[0:64, :])
    val1 = jnp.where(lhanded, vb[64:128, :], vb[0:64, :])
    bad = jnp.where(lhanded, mk[64:128, :], mk[0:64, :]) > 0.5
    sgn0 = jnp.where(lhanded, -1.0, 1.0)

    r0 = (val0 - sgn0 * m0) * inv_s
    r1 = (val1 - m1) * inv_s
    r0 = jnp.where(bad, 0.0, r0)
    r1 = jnp.where(bad, 0.0, r1)
    out_ref[0, :, :] = r0[:61]
    out_ref[1, :, :] = r1[:61]


@jax.jit
def kernel(frames):
    S0, S1, Sm = _build_sel()
    xT = jnp.transpose(frames, (2, 1, 0))

    FA = 1024
    a_specs = [
        pl.BlockSpec((3, _RB, FA), lambda fb, b=b: (0, b, fb))
        for b in _STAT_ROWBLKS
    ]
    partials, handcnt = pl.pallas_call(
        _stats_body,
        grid=(N_FRAMES // FA,),
        in_specs=[
            pl.BlockSpec((3, _NRB * _RB, 1), lambda fb: (0, 0, 0)),
            *a_specs,
        ],
        out_specs=[
            pl.BlockSpec((1, 128), lambda fb: (0, 0)),
            pl.BlockSpec((2, FA), lambda fb: (0, fb)),
        ],
        out_shape=[
            jax.ShapeDtypeStruct((1, 128), jnp.float32),
            jax.ShapeDtypeStruct((2, N_FRAMES), jnp.float32),
        ],
    )(jnp.asarray(_build_stat_masks()), *([xT] * _NRB))

    FB = 1024
    E = np.zeros((FB, FB // 2), np.float32)
    E[np.arange(0, FB, 2), np.arange(FB // 2)] = 1.0
    E = jnp.asarray(E)

    x_specs = [
        pl.BlockSpec((3, _RB, FB), lambda j, b=b: (0, b, j))
        for b in _SEL_ROWBLKS
    ]
    full = pl.pallas_call(
        _main_body,
        grid=(N_FRAMES // FB,),
        in_specs=[
            pl.BlockSpec((1, 128), lambda j: (0, 0)),
            pl.BlockSpec((2, N_FRAMES), lambda j: (0, 0)),
            *x_specs,
            pl.BlockSpec((128, _NSB * _RB), lambda j: (0, 0)),
            pl.BlockSpec((128, _NSB * _RB), lambda j: (0, 0)),
            pl.BlockSpec((128, _NSB * _RB), lambda j: (0, 0)),
            pl.BlockSpec((FB, FB // 2), lambda j: (0, 0)),
        ],
        out_specs=pl.BlockSpec((2, 61, FB // 2), lambda j: (0, 0, j)),
        out_shape=jax.ShapeDtypeStruct((2, 61, MAX_LEN), jnp.float32),
    )(partials, handcnt, *([xT] * _NSB), S0, S1, Sm, E)

    return jnp.transpose(full, (2, 1, 0))
